# baseline (speedup 1.0000x reference)
import functools

import jax
import jax.numpy as jnp
from jax import lax
from jax.experimental import pallas as pl
from jax.experimental.pallas import tpu as pltpu
from jax.experimental.pallas import tpu_sc as plsc

_G = 512
_K = 32


def _fps_body(x_ref, y_ref, z_ref, md_in_ref, cx_ref, cy_ref, cz_ref,
              md_ref):
    B, N = x_ref.shape
    G = cx_ref.shape[1]
    x = x_ref[...]
    y = y_ref[...]
    z = z_ref[...]
    glanes = lax.broadcasted_iota(jnp.int32, (B, G), 1)
    md_ref[...] = md_in_ref[...]

    def body(i, _):
        md = md_ref[...]
        tx, ty, tz = x, y, z
        width = N
        while width > 1:
            width //= 2
            lo = md[:, :width]
            hi = md[:, width:]
            take = hi > lo
            md = jnp.where(take, hi, lo)
            tx = jnp.where(take, tx[:, width:], tx[:, :width])
            ty = jnp.where(take, ty[:, width:], ty[:, :width])
            tz = jnp.where(take, tz[:, width:], tz[:, :width])
        cx, cy, cz = tx, ty, tz
        gm = glanes == i
        cx_ref[...] = jnp.where(gm, cx, cx_ref[...])
        cy_ref[...] = jnp.where(gm, cy, cy_ref[...])
        cz_ref[...] = jnp.where(gm, cz, cz_ref[...])
        dx = x - cx
        dy = y - cy
        dz = z - cz
        d = dx * dx + dy * dy + dz * dz
        md_ref[...] = jnp.minimum(md_ref[...], d)
        return 0

    lax.fori_loop(0, G, body, 0)


def _fps(x, y, z, md, g):
    B, N = x.shape
    out = jax.ShapeDtypeStruct((B, g), jnp.float32)
    mdo = jax.ShapeDtypeStruct((B, N), jnp.float32)
    return pl.pallas_call(
        _fps_body,
        out_shape=[out, out, out, mdo],
    )(x, y, z, md)


_SEG = 64
_CPS = _SEG // 16
_BLK = 256
_SPB = _BLK // _SEG


def _knn_gather_sc(x, y, z, w, cx, cy, cz, g, k):
    B, N = x.shape
    nw = 32
    rows_per_w = (B * g) // nw
    w_per_b = nw // B
    nseg = N // _SEG
    per_w = rows_per_w * k * 4
    mesh = plsc.VectorSubcoreMesh(core_axis_name="c", subcore_axis_name="s")
    big = jnp.int32(2 ** 30)
    inf = jnp.float32(jnp.inf)

    @functools.partial(
        pl.kernel,
        out_type=jax.ShapeDtypeStruct((nw * per_w,), jnp.float32),
        mesh=mesh,
        compiler_params=pltpu.CompilerParams(needs_layout_passes=False),
        scratch_types=[
            pltpu.VMEM((N,), jnp.float32),
            pltpu.VMEM((N,), jnp.float32),
            pltpu.VMEM((N,), jnp.float32),
            pltpu.VMEM((N,), jnp.float32),
            pltpu.VMEM((rows_per_w,), jnp.float32),
            pltpu.VMEM((rows_per_w,), jnp.float32),
            pltpu.VMEM((rows_per_w,), jnp.float32),
            pltpu.VMEM((N,), jnp.float32),
            pltpu.VMEM((N,), jnp.float32),
            pltpu.VMEM((nseg,), jnp.float32),
            pltpu.VMEM((nseg,), jnp.float32),
            pltpu.VMEM((16,), jnp.float32),
            pltpu.VMEM((16,), jnp.float32),
            pltpu.VMEM((k * 4,), jnp.int32),
            pltpu.VMEM((per_w,), jnp.float32),
        ],
    )
    def sc_knn(x_hbm, y_hbm, z_hbm, w_hbm, cx_hbm, cy_hbm, cz_hbm, out_hbm,
               x_v, y_v, z_v, w_v, cx_v, cy_v, cz_v, d2a_v, d2b_v, sma_v,
               smb_v, ssma_v, ssmb_v, sel_v, out_v):
        wid = lax.axis_index("s") * 2 + lax.axis_index("c")
        b = wid // w_per_b
        g0 = (wid % w_per_b) * rows_per_w
        pltpu.sync_copy(x_hbm.at[b], x_v)
        pltpu.sync_copy(y_hbm.at[b], y_v)
        pltpu.sync_copy(z_hbm.at[b], z_v)
        pltpu.sync_copy(w_hbm.at[b], w_v)
        pltpu.sync_copy(cx_hbm.at[b, pl.ds(g0, rows_per_w)], cx_v)
        pltpu.sync_copy(cy_hbm.at[b, pl.ds(g0, rows_per_w)], cy_v)
        pltpu.sync_copy(cz_hbm.at[b, pl.ds(g0, rows_per_w)], cz_v)

        lane = jax.lax.iota(jnp.int32, 16)
        feat = lane % 4
        lane0 = lane == 0

        def splat_at(ref, i):
            return plsc.load_gather(ref, [jnp.full((16,), i, jnp.int32)])

        def store_at(ref, i, v):
            plsc.store_scatter(ref, [jnp.full((16,), i, jnp.int32)],
                               jnp.full((16,), v), mask=lane0)

        def select_and_emit(r, d2_v, sm_v, ssm_v, cxr, cyr, czr):
            for t in range(nseg // 16):
                store_at(ssm_v, t, jnp.min(sm_v[pl.ds(t * 16, 16)]))

            def ext_body(e, _):
                sv = ssm_v[pl.ds(0, 16)]
                smin = jnp.min(sv)
                su = jnp.min(jnp.where(sv == smin, lane, big))
                v = sm_v[pl.ds(su * 16, 16)]
                mval = jnp.min(v)
                s = su * 16 + jnp.min(jnp.where(v == mval, lane, big))

                cbv = jnp.full((16,), inf)
                cbi = jnp.full((16,), big)
                for j in range(_CPS):
                    base = s * _SEG + j * 16
                    v = d2_v[pl.ds(base, 16)]
                    ids = lane + base
                    take = v < cbv
                    cbv = jnp.where(take, v, cbv)
                    cbi = jnp.where(take, ids, cbi)
                cmval = jnp.min(cbv)
                idx = jnp.min(jnp.where(cbv == cmval, cbi, big))

                plsc.store_scatter(sel_v, [e * 4 + lane],
                                   jnp.full((16,), idx), mask=lane < 4)

                store_at(d2_v, idx, inf)
                m = jnp.full((16,), inf)
                for j in range(_CPS):
                    m = jnp.minimum(m, d2_v[pl.ds(s * _SEG + j * 16, 16)])
                store_at(sm_v, s, jnp.min(m))
                store_at(ssm_v, su, jnp.min(sm_v[pl.ds(su * 16, 16)]))
                return 0

            lax.fori_loop(0, k, ext_body, 0, unroll=False)

            cc = jnp.where(feat == 0, cxr,
                           jnp.where(feat == 1, cyr,
                                     jnp.where(feat == 2, czr, 0.0)))

            def out_body(j, _):
                ids = sel_v[pl.ds(j * 16, 16)]
                vx = plsc.load_gather(x_v, [ids])
                vy = plsc.load_gather(y_v, [ids])
                vz = plsc.load_gather(z_v, [ids])
                vw = plsc.load_gather(w_v, [ids])
                val = jnp.where(feat == 0, vx,
                                jnp.where(feat == 1, vy,
                                          jnp.where(feat == 2, vz, vw)))
                out_v[pl.ds(r * (k * 4) + j * 16, 16)] = val - cc
                return 0

            lax.fori_loop(0, k * 4 // 16, out_body, 0, unroll=True)

        def row_body(r2, _):
            r0 = r2 * 2
            cs = []
            for q in range(2):
                cs.append((splat_at(cx_v, r0 + q), splat_at(cy_v, r0 + q),
                           splat_at(cz_v, r0 + q)))
            d2s = (d2a_v, d2b_v)
            sms = (sma_v, smb_v)
            ssms = (ssma_v, ssmb_v)

            def seg_body(s, _):
                ms = [[jnp.full((16,), inf) for _ in range(_SPB)]
                      for _ in range(2)]
                for j in range(_BLK // 16):
                    base = s * _BLK + j * 16
                    xv = x_v[pl.ds(base, 16)]
                    yv = y_v[pl.ds(base, 16)]
                    zv = z_v[pl.ds(base, 16)]
                    for q in range(2):
                        cxq, cyq, czq = cs[q]
                        dx = cxq - xv
                        dy = cyq - yv
                        dz = czq - zv
                        d2 = dx * dx + dy * dy + dz * dz
                        d2s[q][pl.ds(base, 16)] = d2
                        ms[q][j // _CPS] = jnp.minimum(ms[q][j // _CPS], d2)
                for q in range(2):
                    for t in range(_SPB):
                        store_at(sms[q], s * _SPB + t, jnp.min(ms[q][t]))
                return 0

            lax.fori_loop(0, N // _BLK, seg_body, 0, unroll=False)

            for q in range(2):
                select_and_emit(r0 + q, d2s[q], sms[q], ssms[q], *cs[q])
            return 0

        lax.fori_loop(0, rows_per_w // 2, row_body, 0, unroll=False)
        pltpu.sync_copy(out_v, out_hbm.at[pl.ds(wid * per_w, per_w)])

    return sc_knn(x, y, z, w, cx, cy, cz)


_STAGES = 8


def kernel(points):
    B, N, C = points.shape
    x = points[:, :, 0]
    y = points[:, :, 1]
    z = points[:, :, 2]
    w = points[:, :, 3]

    gs = _G // _STAGES
    md = jnp.full((B, N), 1e10, jnp.float32)
    group_parts = []
    center_parts = []
    for _ in range(_STAGES):
        cx, cy, cz, md = _fps(x, y, z, md, gs)
        center_parts.append(jnp.stack([cx, cy, cz], axis=-1))
        flat = _knn_gather_sc(x, y, z, w, cx, cy, cz, gs, _K)
        group_parts.append(flat.reshape(B, gs, _K, 4))
    groups = jnp.concatenate(group_parts, axis=1)
    centers = jnp.concatenate(center_parts, axis=1)
    return groups, centers


# speedup vs baseline: 1.0813x; 1.0813x over previous
import functools

import jax
import jax.numpy as jnp
from jax import lax
from jax.experimental import pallas as pl
from jax.experimental.pallas import tpu as pltpu
from jax.experimental.pallas import tpu_sc as plsc

_G = 512
_K = 32


def _fps_body(x_ref, y_ref, z_ref, md_in_ref, cx_ref, cy_ref, cz_ref,
              md_ref):
    B, N = x_ref.shape
    G = cx_ref.shape[1]
    x = x_ref[...]
    y = y_ref[...]
    z = z_ref[...]
    glanes = lax.broadcasted_iota(jnp.int32, (B, G), 1)
    md_ref[...] = md_in_ref[...]

    def body(i, _):
        md = md_ref[...]
        tx, ty, tz = x, y, z
        width = N
        while width > 1:
            width //= 2
            lo = md[:, :width]
            hi = md[:, width:]
            take = hi > lo
            md = jnp.where(take, hi, lo)
            tx = jnp.where(take, tx[:, width:], tx[:, :width])
            ty = jnp.where(take, ty[:, width:], ty[:, :width])
            tz = jnp.where(take, tz[:, width:], tz[:, :width])
        cx, cy, cz = tx, ty, tz
        gm = glanes == i
        cx_ref[...] = jnp.where(gm, cx, cx_ref[...])
        cy_ref[...] = jnp.where(gm, cy, cy_ref[...])
        cz_ref[...] = jnp.where(gm, cz, cz_ref[...])
        dx = x - cx
        dy = y - cy
        dz = z - cz
        d = dx * dx + dy * dy + dz * dz
        md_ref[...] = jnp.minimum(md_ref[...], d)
        return 0

    lax.fori_loop(0, G, body, 0)


def _fps(x, y, z, md, g):
    B, N = x.shape
    out = jax.ShapeDtypeStruct((B, g), jnp.float32)
    mdo = jax.ShapeDtypeStruct((B, N), jnp.float32)
    return pl.pallas_call(
        _fps_body,
        out_shape=[out, out, out, mdo],
    )(x, y, z, md)


_SEG = 64
_CPS = _SEG // 16
_BLK = 256
_SPB = _BLK // _SEG


def _knn_gather_sc(x, y, z, w, cx, cy, cz, g, k):
    B, N = x.shape
    nw = 32
    rows_per_w = (B * g) // nw
    w_per_b = nw // B
    nseg = N // _SEG
    per_w = rows_per_w * k * 4
    mesh = plsc.VectorSubcoreMesh(core_axis_name="c", subcore_axis_name="s")
    big = jnp.int32(2 ** 30)
    inf = jnp.float32(jnp.inf)

    @functools.partial(
        pl.kernel,
        out_type=jax.ShapeDtypeStruct((nw * per_w,), jnp.float32),
        mesh=mesh,
        compiler_params=pltpu.CompilerParams(needs_layout_passes=False),
        scratch_types=[
            pltpu.VMEM((N,), jnp.float32),
            pltpu.VMEM((N,), jnp.float32),
            pltpu.VMEM((N,), jnp.float32),
            pltpu.VMEM((N,), jnp.float32),
            pltpu.VMEM((rows_per_w,), jnp.float32),
            pltpu.VMEM((rows_per_w,), jnp.float32),
            pltpu.VMEM((rows_per_w,), jnp.float32),
            pltpu.VMEM((N,), jnp.float32),
            pltpu.VMEM((N,), jnp.float32),
            pltpu.VMEM((nseg,), jnp.float32),
            pltpu.VMEM((nseg,), jnp.float32),
            pltpu.VMEM((k * 4,), jnp.int32),
            pltpu.VMEM((per_w,), jnp.float32),
        ],
    )
    def sc_knn(x_hbm, y_hbm, z_hbm, w_hbm, cx_hbm, cy_hbm, cz_hbm, out_hbm,
               x_v, y_v, z_v, w_v, cx_v, cy_v, cz_v, d2a_v, d2b_v, sma_v,
               smb_v, sel_v, out_v):
        wid = lax.axis_index("s") * 2 + lax.axis_index("c")
        b = wid // w_per_b
        g0 = (wid % w_per_b) * rows_per_w
        pltpu.sync_copy(x_hbm.at[b], x_v)
        pltpu.sync_copy(y_hbm.at[b], y_v)
        pltpu.sync_copy(z_hbm.at[b], z_v)
        pltpu.sync_copy(w_hbm.at[b], w_v)
        pltpu.sync_copy(cx_hbm.at[b, pl.ds(g0, rows_per_w)], cx_v)
        pltpu.sync_copy(cy_hbm.at[b, pl.ds(g0, rows_per_w)], cy_v)
        pltpu.sync_copy(cz_hbm.at[b, pl.ds(g0, rows_per_w)], cz_v)

        lane = jax.lax.iota(jnp.int32, 16)
        feat = lane % 4
        lane0 = lane == 0

        def splat_at(ref, i):
            return plsc.load_gather(ref, [jnp.full((16,), i, jnp.int32)])

        def store_at(ref, i, v):
            plsc.store_scatter(ref, [jnp.full((16,), i, jnp.int32)],
                               jnp.full((16,), v), mask=lane0)

        def select_and_emit(r, d2_v, sm_v, cxr, cyr, czr):
            def ext_body(e, _):
                bv = jnp.full((16,), inf)
                bi = jnp.full((16,), big)
                for j in range(nseg // 16):
                    v = sm_v[pl.ds(j * 16, 16)]
                    ids = lane + j * 16
                    take = v < bv
                    bv = jnp.where(take, v, bv)
                    bi = jnp.where(take, ids, bi)
                mval = jnp.min(bv)
                s = jnp.min(jnp.where(bv == mval, bi, big))

                cbv = jnp.full((16,), inf)
                cbi = jnp.full((16,), big)
                for j in range(_CPS):
                    base = s * _SEG + j * 16
                    v = d2_v[pl.ds(base, 16)]
                    ids = lane + base
                    take = v < cbv
                    cbv = jnp.where(take, v, cbv)
                    cbi = jnp.where(take, ids, cbi)
                cmval = jnp.min(cbv)
                idx = jnp.min(jnp.where(cbv == cmval, cbi, big))

                plsc.store_scatter(sel_v, [e * 4 + lane],
                                   jnp.full((16,), idx), mask=lane < 4)

                store_at(d2_v, idx, inf)
                m = jnp.full((16,), inf)
                for j in range(_CPS):
                    m = jnp.minimum(m, d2_v[pl.ds(s * _SEG + j * 16, 16)])
                store_at(sm_v, s, jnp.min(m))
                return 0

            lax.fori_loop(0, k, ext_body, 0, unroll=False)

            cc = jnp.where(feat == 0, cxr,
                           jnp.where(feat == 1, cyr,
                                     jnp.where(feat == 2, czr, 0.0)))

            def out_body(j, _):
                ids = sel_v[pl.ds(j * 16, 16)]
                vx = plsc.load_gather(x_v, [ids])
                vy = plsc.load_gather(y_v, [ids])
                vz = plsc.load_gather(z_v, [ids])
                vw = plsc.load_gather(w_v, [ids])
                val = jnp.where(feat == 0, vx,
                                jnp.where(feat == 1, vy,
                                          jnp.where(feat == 2, vz, vw)))
                out_v[pl.ds(r * (k * 4) + j * 16, 16)] = val - cc
                return 0

            lax.fori_loop(0, k * 4 // 16, out_body, 0, unroll=True)

        def row_body(r2, _):
            r0 = r2 * 2
            cs = []
            for q in range(2):
                cs.append((splat_at(cx_v, r0 + q), splat_at(cy_v, r0 + q),
                           splat_at(cz_v, r0 + q)))
            d2s = (d2a_v, d2b_v)
            sms = (sma_v, smb_v)

            def seg_body(s, _):
                ms = [[jnp.full((16,), inf) for _ in range(_SPB)]
                      for _ in range(2)]
                for j in range(_BLK // 16):
                    base = s * _BLK + j * 16
                    xv = x_v[pl.ds(base, 16)]
                    yv = y_v[pl.ds(base, 16)]
                    zv = z_v[pl.ds(base, 16)]
                    for q in range(2):
                        cxq, cyq, czq = cs[q]
                        dx = cxq - xv
                        dy = cyq - yv
                        dz = czq - zv
                        d2 = dx * dx + dy * dy + dz * dz
                        d2s[q][pl.ds(base, 16)] = d2
                        ms[q][j // _CPS] = jnp.minimum(ms[q][j // _CPS], d2)
                for q in range(2):
                    for t in range(_SPB):
                        store_at(sms[q], s * _SPB + t, jnp.min(ms[q][t]))
                return 0

            lax.fori_loop(0, N // _BLK, seg_body, 0, unroll=False)

            for q in range(2):
                select_and_emit(r0 + q, d2s[q], sms[q], *cs[q])
            return 0

        lax.fori_loop(0, rows_per_w // 2, row_body, 0, unroll=False)
        pltpu.sync_copy(out_v, out_hbm.at[pl.ds(wid * per_w, per_w)])

    return sc_knn(x, y, z, w, cx, cy, cz)


_STAGES = 8


def kernel(points):
    B, N, C = points.shape
    x = points[:, :, 0]
    y = points[:, :, 1]
    z = points[:, :, 2]
    w = points[:, :, 3]

    gs = _G // _STAGES
    md = jnp.full((B, N), 1e10, jnp.float32)
    group_parts = []
    center_parts = []
    for _ in range(_STAGES):
        cx, cy, cz, md = _fps(x, y, z, md, gs)
        center_parts.append(jnp.stack([cx, cy, cz], axis=-1))
        flat = _knn_gather_sc(x, y, z, w, cx, cy, cz, gs, _K)
        group_parts.append(flat.reshape(B, gs, _K, 4))
    groups = jnp.concatenate(group_parts, axis=1)
    centers = jnp.concatenate(center_parts, axis=1)
    return groups, centers
